# baseline (speedup 1.0000x reference)
import jax
import jax.numpy as jnp
from jax import lax
from jax.experimental import pallas as pl
from jax.experimental.pallas import tpu as pltpu
from jax.experimental.pallas import tpu_sc as plsc

_OUT = 7
_C = 256
_BOXES_PER_W = 16
_NROW = 25088


_GATHER_DN = lax.GatherDimensionNumbers(
    offset_dims=(), collapsed_slice_dims=(0,), start_index_map=(0,))


def _splat16(v, i):
    iv = (jnp.zeros((16,), jnp.int32) + i)[:, None]
    return lax.gather(v, iv, _GATHER_DN, (1,),
                      mode=lax.GatherScatterMode.PROMISE_IN_BOUNDS)


def _sc_body(table, px1, py1, pbw, pbh, pbase, pwid, out,
             x1v, y1v, bwv, bhv, basev, wv,
             idxb0, idxb1, wtb0, wtb1, gbuf0, gbuf1, obuf, sem0, sem1):
    wid = lax.axis_index("s") * 2 + lax.axis_index("c")
    b0 = wid * _BOXES_PER_W

    pltpu.sync_copy(px1.at[pl.ds(b0, 16)], x1v)
    pltpu.sync_copy(py1.at[pl.ds(b0, 16)], y1v)
    pltpu.sync_copy(pbw.at[pl.ds(b0, 16)], bwv)
    pltpu.sync_copy(pbh.at[pl.ds(b0, 16)], bhv)
    pltpu.sync_copy(pbase.at[pl.ds(b0, 16)], basev)
    pltpu.sync_copy(pwid.at[pl.ds(b0, 16)], wv)
    x1a = x1v[...]
    y1a = y1v[...]
    bwa = bwv[...]
    bha = bhv[...]
    basea = basev[...]
    wa = wv[...]

    ii = lax.iota(jnp.int32, 16)
    pos = ((ii >> 1).astype(jnp.float32)
           + (ii & 1).astype(jnp.float32) * 0.5 + 0.25)

    def corners(v, wm1, wm1f):
        v = jnp.maximum(v, 0.0)
        lo = v.astype(jnp.int32)
        cl = lo >= wm1
        lo = jnp.where(cl, wm1, lo)
        hi = jnp.where(cl, wm1, lo + 1)
        vf = jnp.where(cl, wm1f, v)
        return lo, hi, vf - lo.astype(jnp.float32)

    def box_body(lb, carry):
        base = _splat16(basea, lb)
        w = _splat16(wa, lb)
        wm1 = w - 1
        wm1f = wm1.astype(jnp.float32)
        xl, xh, lx = corners(_splat16(x1a, lb) + pos * _splat16(bwa, lb),
                             wm1, wm1f)
        yl, yh, ly = corners(_splat16(y1a, lb) + pos * _splat16(bha, lb),
                             wm1, wm1f)
        hx = 1.0 - lx

        def build_row(pyr, idxb, wtb):
            for il in range(2):
                i = 2 * pyr + il
                yl_s = _splat16(yl, i)
                yh_s = _splat16(yh, i)
                ly_s = _splat16(ly, i)
                hy_s = 1.0 - ly_s
                for cy in range(2):
                    rb = base + (yl_s if cy == 0 else yh_s) * w
                    wy = (hy_s if cy == 0 else ly_s) * 0.25
                    for cx in range(2):
                        q = (cy * 2 + cx) * 2 + il
                        idxb[pl.ds(q * 16, 16)] = rb + (xl if cx == 0 else xh)
                        wtb[q, :] = wy * (hx if cx == 0 else lx)

        def fire(idxb, gbuf, sem):
            pltpu.make_async_copy(table.at[idxb], gbuf, sem).start()

        def wait(idxb, gbuf, sem):
            pltpu.make_async_copy(table.at[idxb], gbuf, sem).wait()

        def combine_row(pyr, gbuf, wtb):
            @plsc.parallel_loop(0, _OUT)
            def px_body(px):
                def u_body(u, acc):
                    il = u >> 1
                    jj = u & 1
                    lane = 2 * px + jj
                    iv = (jnp.zeros((16,), jnp.int32) + lane)[:, None]
                    nb = il * 16 + lane
                    new = list(acc)
                    for c in range(4):
                        wgt = lax.gather(
                            wtb[c * 2 + il, :], iv, _GATHER_DN, (1,),
                            mode=lax.GatherScatterMode.PROMISE_IN_BOUNDS)
                        n = nb + c * 32
                        for g in range(8):
                            v = gbuf[n, pl.ds(g * 16, 16)]
                            lo = lax.bitcast_convert_type(
                                jnp.left_shift(v, 16), jnp.float32)
                            hi = lax.bitcast_convert_type(v, jnp.float32)
                            new[g] = new[g] + wgt * lo
                            new[8 + g] = new[8 + g] + wgt * hi
                    return tuple(new)
                acc0 = tuple(jnp.zeros((16,), jnp.float32) for _ in range(16))
                acc = lax.fori_loop(0, 4, u_body, acc0)
                for g in range(8):
                    obuf[pyr * _OUT + px, pl.ds(g * 16, 16)] = acc[g]
                    obuf[pyr * _OUT + px, pl.ds(128 + g * 16, 16)] = acc[8 + g]

        build_row(0, idxb0, wtb0)
        fire(idxb0, gbuf0, sem0)

        def two_rows(kk, carry2):
            c0 = 2 * kk
            build_row(c0 + 1, idxb1, wtb1)
            fire(idxb1, gbuf1, sem1)
            wait(idxb0, gbuf0, sem0)
            combine_row(c0, gbuf0, wtb0)
            build_row(c0 + 2, idxb0, wtb0)
            fire(idxb0, gbuf0, sem0)
            wait(idxb1, gbuf1, sem1)
            combine_row(c0 + 1, gbuf1, wtb1)
            return carry2
        lax.fori_loop(0, 3, two_rows, 0)
        wait(idxb0, gbuf0, sem0)
        combine_row(6, gbuf0, wtb0)
        pltpu.sync_copy(obuf, out.at[b0 + lb])
        return carry
    lax.fori_loop(0, _BOXES_PER_W, box_body, 0)


def kernel(x0, x1, x2, x3, boxes0, boxes1):
    feats = (x0, x1, x2, x3)
    table = jnp.concatenate(
        [jnp.transpose(f, (0, 2, 3, 1)).reshape(-1, _C) for f in feats],
        axis=0)

    def _rtne(f32):
        u = lax.bitcast_convert_type(f32, jnp.uint32)
        return (u + 0x7FFF + ((u >> 16) & 1)) >> 16

    table = lax.bitcast_convert_type(
        _rtne(table[:, :128]) | (_rtne(table[:, 128:]) << 16), jnp.int32)

    boxes = jnp.concatenate([boxes0, boxes1], axis=0)
    nb0 = boxes0.shape[0]
    b_idx = jnp.concatenate([
        jnp.zeros((nb0,), jnp.int32),
        jnp.ones((boxes.shape[0] - nb0,), jnp.int32)])
    areas = (boxes[:, 2] - boxes[:, 0]) * (boxes[:, 3] - boxes[:, 1])
    sizes = jnp.sqrt(areas)
    lvl = jnp.floor(4.0 + jnp.log2(sizes / 224.0 + 1e-8))
    k = jnp.clip(lvl, 2, 5).astype(jnp.int32) - 2
    scale = jnp.take(jnp.array((0.25, 0.125, 0.0625, 0.03125), jnp.float32), k)
    wlv = jnp.take(jnp.array([128, 64, 32, 16], jnp.int32), k)
    lvl_off = jnp.take(jnp.array([0, 32768, 40960, 43008], jnp.int32), k)
    base = lvl_off + b_idx * wlv * wlv
    bx1 = boxes[:, 0] * scale - 0.5
    by1 = boxes[:, 1] * scale - 0.5
    bx2 = boxes[:, 2] * scale - 0.5
    by2 = boxes[:, 3] * scale - 0.5
    bw = (bx2 - bx1) / _OUT
    bh = (by2 - by1) / _OUT

    mesh = plsc.VectorSubcoreMesh(core_axis_name="c", subcore_axis_name="s")
    out_t = pl.kernel(
        _sc_body,
        out_type=jax.ShapeDtypeStruct((512, 49, _C), jnp.float32),
        mesh=mesh,
        scratch_types=[
            pltpu.VMEM((16,), jnp.float32),
            pltpu.VMEM((16,), jnp.float32),
            pltpu.VMEM((16,), jnp.float32),
            pltpu.VMEM((16,), jnp.float32),
            pltpu.VMEM((16,), jnp.int32),
            pltpu.VMEM((16,), jnp.int32),
            pltpu.VMEM((128,), jnp.int32),
            pltpu.VMEM((128,), jnp.int32),
            pltpu.VMEM((8, 16), jnp.float32),
            pltpu.VMEM((8, 16), jnp.float32),
            pltpu.VMEM((128, _C // 2), jnp.int32),
            pltpu.VMEM((128, _C // 2), jnp.int32),
            pltpu.VMEM((49, _C), jnp.float32),
            pltpu.SemaphoreType.DMA,
            pltpu.SemaphoreType.DMA,
        ],
    )(table, bx1, by1, bw, bh, base, wlv)

    return out_t.reshape(512, _OUT, _OUT, _C).transpose(0, 3, 1, 2)


# speedup vs baseline: 1.0004x; 1.0004x over previous
import jax
import jax.numpy as jnp
from jax import lax
from jax.experimental import pallas as pl
from jax.experimental.pallas import tpu as pltpu
from jax.experimental.pallas import tpu_sc as plsc

_OUT = 7
_C = 256
_BOXES_PER_W = 16
_NROW = 25088


_GATHER_DN = lax.GatherDimensionNumbers(
    offset_dims=(), collapsed_slice_dims=(0,), start_index_map=(0,))


def _splat16(v, i):
    iv = (jnp.zeros((16,), jnp.int32) + i)[:, None]
    return lax.gather(v, iv, _GATHER_DN, (1,),
                      mode=lax.GatherScatterMode.PROMISE_IN_BOUNDS)


def _sc_body(table, px1, py1, pbw, pbh, pbase, pwid, out,
             x1v, y1v, bwv, bhv, basev, wv,
             idxb0, idxb1, wtb0, wtb1, gbuf0, gbuf1, obuf, sem0, sem1):
    wid = lax.axis_index("s") * 2 + lax.axis_index("c")
    b0 = wid * _BOXES_PER_W

    pltpu.sync_copy(px1.at[pl.ds(b0, 16)], x1v)
    pltpu.sync_copy(py1.at[pl.ds(b0, 16)], y1v)
    pltpu.sync_copy(pbw.at[pl.ds(b0, 16)], bwv)
    pltpu.sync_copy(pbh.at[pl.ds(b0, 16)], bhv)
    pltpu.sync_copy(pbase.at[pl.ds(b0, 16)], basev)
    pltpu.sync_copy(pwid.at[pl.ds(b0, 16)], wv)
    x1a = x1v[...]
    y1a = y1v[...]
    bwa = bwv[...]
    bha = bhv[...]
    basea = basev[...]
    wa = wv[...]

    ii = lax.iota(jnp.int32, 16)
    pos = ((ii >> 1).astype(jnp.float32)
           + (ii & 1).astype(jnp.float32) * 0.5 + 0.25)

    def corners(v, wm1, wm1f):
        v = jnp.maximum(v, 0.0)
        lo = v.astype(jnp.int32)
        cl = lo >= wm1
        lo = jnp.where(cl, wm1, lo)
        hi = jnp.where(cl, wm1, lo + 1)
        vf = jnp.where(cl, wm1f, v)
        return lo, hi, vf - lo.astype(jnp.float32)

    def box_body(lb, carry):
        base = _splat16(basea, lb)
        w = _splat16(wa, lb)
        wm1 = w - 1
        wm1f = wm1.astype(jnp.float32)
        xl, xh, lx = corners(_splat16(x1a, lb) + pos * _splat16(bwa, lb),
                             wm1, wm1f)
        yl, yh, ly = corners(_splat16(y1a, lb) + pos * _splat16(bha, lb),
                             wm1, wm1f)
        hx = 1.0 - lx

        def build_row(pyr, idxb, wtb):
            for il in range(2):
                i = 2 * pyr + il
                yl_s = _splat16(yl, i)
                yh_s = _splat16(yh, i)
                ly_s = _splat16(ly, i)
                hy_s = 1.0 - ly_s
                for cy in range(2):
                    rb = base + (yl_s if cy == 0 else yh_s) * w
                    wy = (hy_s if cy == 0 else ly_s) * 0.25
                    for cx in range(2):
                        q = (cy * 2 + cx) * 2 + il
                        idxb[pl.ds(q * 16, 16)] = rb + (xl if cx == 0 else xh)
                        wtb[q, :] = wy * (hx if cx == 0 else lx)

        def fire(idxb, gbuf, sem):
            pltpu.make_async_copy(table.at[idxb], gbuf, sem).start()

        def wait(idxb, gbuf, sem):
            pltpu.make_async_copy(table.at[idxb], gbuf, sem).wait()

        def combine_row(pyr, gbuf, wtb):
            def px_body(px, carry3):
                def u_body(u, acc):
                    il = u >> 1
                    jj = u & 1
                    lane = 2 * px + jj
                    iv = (jnp.zeros((16,), jnp.int32) + lane)[:, None]
                    nb = il * 16 + lane
                    new = list(acc)
                    for c in range(4):
                        wgt = lax.gather(
                            wtb[c * 2 + il, :], iv, _GATHER_DN, (1,),
                            mode=lax.GatherScatterMode.PROMISE_IN_BOUNDS)
                        n = nb + c * 32
                        for g in range(8):
                            v = gbuf[n, pl.ds(g * 16, 16)]
                            lo = lax.bitcast_convert_type(
                                jnp.left_shift(v, 16), jnp.float32)
                            hi = lax.bitcast_convert_type(v, jnp.float32)
                            new[g] = new[g] + wgt * lo
                            new[8 + g] = new[8 + g] + wgt * hi
                    return tuple(new)
                acc0 = tuple(jnp.zeros((16,), jnp.float32) for _ in range(16))
                acc = lax.fori_loop(0, 4, u_body, acc0)
                for g in range(8):
                    obuf[pyr * _OUT + px, pl.ds(g * 16, 16)] = acc[g]
                    obuf[pyr * _OUT + px, pl.ds(128 + g * 16, 16)] = acc[8 + g]
                return carry3
            lax.fori_loop(0, _OUT, px_body, 0)

        build_row(0, idxb0, wtb0)
        fire(idxb0, gbuf0, sem0)

        def two_rows(kk, carry2):
            c0 = 2 * kk
            build_row(c0 + 1, idxb1, wtb1)
            fire(idxb1, gbuf1, sem1)
            wait(idxb0, gbuf0, sem0)
            combine_row(c0, gbuf0, wtb0)
            build_row(c0 + 2, idxb0, wtb0)
            fire(idxb0, gbuf0, sem0)
            wait(idxb1, gbuf1, sem1)
            combine_row(c0 + 1, gbuf1, wtb1)
            return carry2
        lax.fori_loop(0, 3, two_rows, 0)
        wait(idxb0, gbuf0, sem0)
        combine_row(6, gbuf0, wtb0)
        pltpu.sync_copy(obuf, out.at[b0 + lb])
        return carry
    lax.fori_loop(0, _BOXES_PER_W, box_body, 0)


def kernel(x0, x1, x2, x3, boxes0, boxes1):
    feats = (x0, x1, x2, x3)
    table = jnp.concatenate(
        [jnp.transpose(f, (0, 2, 3, 1)).reshape(-1, _C) for f in feats],
        axis=0)

    def _rtne(f32):
        u = lax.bitcast_convert_type(f32, jnp.uint32)
        return (u + 0x7FFF + ((u >> 16) & 1)) >> 16

    table = lax.bitcast_convert_type(
        _rtne(table[:, :128]) | (_rtne(table[:, 128:]) << 16), jnp.int32)

    boxes = jnp.concatenate([boxes0, boxes1], axis=0)
    nb0 = boxes0.shape[0]
    b_idx = jnp.concatenate([
        jnp.zeros((nb0,), jnp.int32),
        jnp.ones((boxes.shape[0] - nb0,), jnp.int32)])
    areas = (boxes[:, 2] - boxes[:, 0]) * (boxes[:, 3] - boxes[:, 1])
    sizes = jnp.sqrt(areas)
    lvl = jnp.floor(4.0 + jnp.log2(sizes / 224.0 + 1e-8))
    k = jnp.clip(lvl, 2, 5).astype(jnp.int32) - 2
    scale = jnp.take(jnp.array((0.25, 0.125, 0.0625, 0.03125), jnp.float32), k)
    wlv = jnp.take(jnp.array([128, 64, 32, 16], jnp.int32), k)
    lvl_off = jnp.take(jnp.array([0, 32768, 40960, 43008], jnp.int32), k)
    base = lvl_off + b_idx * wlv * wlv
    bx1 = boxes[:, 0] * scale - 0.5
    by1 = boxes[:, 1] * scale - 0.5
    bx2 = boxes[:, 2] * scale - 0.5
    by2 = boxes[:, 3] * scale - 0.5
    bw = (bx2 - bx1) / _OUT
    bh = (by2 - by1) / _OUT

    mesh = plsc.VectorSubcoreMesh(core_axis_name="c", subcore_axis_name="s")
    out_t = pl.kernel(
        _sc_body,
        out_type=jax.ShapeDtypeStruct((512, 49, _C), jnp.float32),
        mesh=mesh,
        scratch_types=[
            pltpu.VMEM((16,), jnp.float32),
            pltpu.VMEM((16,), jnp.float32),
            pltpu.VMEM((16,), jnp.float32),
            pltpu.VMEM((16,), jnp.float32),
            pltpu.VMEM((16,), jnp.int32),
            pltpu.VMEM((16,), jnp.int32),
            pltpu.VMEM((128,), jnp.int32),
            pltpu.VMEM((128,), jnp.int32),
            pltpu.VMEM((8, 16), jnp.float32),
            pltpu.VMEM((8, 16), jnp.float32),
            pltpu.VMEM((128, _C // 2), jnp.int32),
            pltpu.VMEM((128, _C // 2), jnp.int32),
            pltpu.VMEM((49, _C), jnp.float32),
            pltpu.SemaphoreType.DMA,
            pltpu.SemaphoreType.DMA,
        ],
    )(table, bx1, by1, bw, bh, base, wlv)

    return out_t.reshape(512, _OUT, _OUT, _C).transpose(0, 3, 1, 2)


# speedup vs baseline: 1.0037x; 1.0033x over previous
import jax
import jax.numpy as jnp
from jax import lax
from jax.experimental import pallas as pl
from jax.experimental.pallas import tpu as pltpu
from jax.experimental.pallas import tpu_sc as plsc

_OUT = 7
_C = 256
_BOXES_PER_W = 16
_NROW = 25088


_GATHER_DN = lax.GatherDimensionNumbers(
    offset_dims=(), collapsed_slice_dims=(0,), start_index_map=(0,))


def _splat16(v, i):
    iv = (jnp.zeros((16,), jnp.int32) + i)[:, None]
    return lax.gather(v, iv, _GATHER_DN, (1,),
                      mode=lax.GatherScatterMode.PROMISE_IN_BOUNDS)


def _sc_body(table, px1, py1, pbw, pbh, pbase, pwid, out,
             x1v, y1v, bwv, bhv, basev, wv,
             idxb0, idxb1, wtb0, wtb1, gbuf0, gbuf1, obuf, sem0, sem1):
    wid = lax.axis_index("s") * 2 + lax.axis_index("c")
    b0 = wid * _BOXES_PER_W

    pltpu.sync_copy(px1.at[pl.ds(b0, 16)], x1v)
    pltpu.sync_copy(py1.at[pl.ds(b0, 16)], y1v)
    pltpu.sync_copy(pbw.at[pl.ds(b0, 16)], bwv)
    pltpu.sync_copy(pbh.at[pl.ds(b0, 16)], bhv)
    pltpu.sync_copy(pbase.at[pl.ds(b0, 16)], basev)
    pltpu.sync_copy(pwid.at[pl.ds(b0, 16)], wv)
    x1a = x1v[...]
    y1a = y1v[...]
    bwa = bwv[...]
    bha = bhv[...]
    basea = basev[...]
    wa = wv[...]

    ii = lax.iota(jnp.int32, 16)
    pos = ((ii >> 1).astype(jnp.float32)
           + (ii & 1).astype(jnp.float32) * 0.5 + 0.25)

    def corners(v, wm1, wm1f):
        v = jnp.maximum(v, 0.0)
        lo = v.astype(jnp.int32)
        cl = lo >= wm1
        lo = jnp.where(cl, wm1, lo)
        hi = jnp.where(cl, wm1, lo + 1)
        vf = jnp.where(cl, wm1f, v)
        return lo, hi, vf - lo.astype(jnp.float32)

    def box_body(lb, carry):
        base = _splat16(basea, lb)
        w = _splat16(wa, lb)
        wm1 = w - 1
        wm1f = wm1.astype(jnp.float32)
        xl, xh, lx = corners(_splat16(x1a, lb) + pos * _splat16(bwa, lb),
                             wm1, wm1f)
        yl, yh, ly = corners(_splat16(y1a, lb) + pos * _splat16(bha, lb),
                             wm1, wm1f)
        hx = 1.0 - lx

        def build_row(pyr, idxb, wtb):
            for il in range(2):
                i = 2 * pyr + il
                yl_s = _splat16(yl, i)
                yh_s = _splat16(yh, i)
                ly_s = _splat16(ly, i)
                hy_s = 1.0 - ly_s
                for cy in range(2):
                    rb = base + (yl_s if cy == 0 else yh_s) * w
                    wy = (hy_s if cy == 0 else ly_s) * 0.25
                    for cx in range(2):
                        q = (cy * 2 + cx) * 2 + il
                        idxb[pl.ds(q * 16, 16)] = rb + (xl if cx == 0 else xh)
                        wtb[q, :] = wy * (hx if cx == 0 else lx)

        def fire(idxb, gbuf, sem):
            pltpu.make_async_copy(table.at[idxb], gbuf, sem).start()

        def wait(idxb, gbuf, sem):
            pltpu.make_async_copy(table.at[idxb], gbuf, sem).wait()

        def combine_row(pyr, gbuf, wtb):
            def px_body(px, carry3):
                def u_body(u, acc):
                    il = u >> 1
                    jj = u & 1
                    lane = 2 * px + jj
                    iv = (jnp.zeros((16,), jnp.int32) + lane)[:, None]
                    nb = il * 16 + lane
                    new = list(acc)
                    for c in range(4):
                        wgt = lax.gather(
                            wtb[c * 2 + il, :], iv, _GATHER_DN, (1,),
                            mode=lax.GatherScatterMode.PROMISE_IN_BOUNDS)
                        n = nb + c * 32
                        for g in range(8):
                            v = gbuf[n, pl.ds(g * 16, 16)]
                            lo = lax.bitcast_convert_type(
                                jnp.left_shift(v, 16), jnp.float32)
                            hi = lax.bitcast_convert_type(
                                v & jnp.int32(-65536), jnp.float32)
                            new[g] = new[g] + wgt * lo
                            new[8 + g] = new[8 + g] + wgt * hi
                    return tuple(new)
                acc0 = tuple(jnp.zeros((16,), jnp.float32) for _ in range(16))
                acc = lax.fori_loop(0, 4, u_body, acc0)
                for g in range(8):
                    obuf[pyr * _OUT + px, pl.ds(g * 16, 16)] = acc[g]
                    obuf[pyr * _OUT + px, pl.ds(128 + g * 16, 16)] = acc[8 + g]
                return carry3
            lax.fori_loop(0, _OUT, px_body, 0)

        build_row(0, idxb0, wtb0)
        fire(idxb0, gbuf0, sem0)

        def two_rows(kk, carry2):
            c0 = 2 * kk
            build_row(c0 + 1, idxb1, wtb1)
            fire(idxb1, gbuf1, sem1)
            wait(idxb0, gbuf0, sem0)
            combine_row(c0, gbuf0, wtb0)
            build_row(c0 + 2, idxb0, wtb0)
            fire(idxb0, gbuf0, sem0)
            wait(idxb1, gbuf1, sem1)
            combine_row(c0 + 1, gbuf1, wtb1)
            return carry2
        lax.fori_loop(0, 3, two_rows, 0)
        wait(idxb0, gbuf0, sem0)
        combine_row(6, gbuf0, wtb0)
        pltpu.sync_copy(obuf, out.at[b0 + lb])
        return carry
    lax.fori_loop(0, _BOXES_PER_W, box_body, 0)


def kernel(x0, x1, x2, x3, boxes0, boxes1):
    feats = (x0, x1, x2, x3)
    table = jnp.concatenate(
        [jnp.transpose(f, (0, 2, 3, 1)).reshape(-1, _C) for f in feats],
        axis=0)

    def _rtne(f32):
        u = lax.bitcast_convert_type(f32, jnp.uint32)
        return (u + 0x7FFF + ((u >> 16) & 1)) >> 16

    table = lax.bitcast_convert_type(
        _rtne(table[:, :128]) | (_rtne(table[:, 128:]) << 16), jnp.int32)

    boxes = jnp.concatenate([boxes0, boxes1], axis=0)
    nb0 = boxes0.shape[0]
    b_idx = jnp.concatenate([
        jnp.zeros((nb0,), jnp.int32),
        jnp.ones((boxes.shape[0] - nb0,), jnp.int32)])
    areas = (boxes[:, 2] - boxes[:, 0]) * (boxes[:, 3] - boxes[:, 1])
    sizes = jnp.sqrt(areas)
    lvl = jnp.floor(4.0 + jnp.log2(sizes / 224.0 + 1e-8))
    k = jnp.clip(lvl, 2, 5).astype(jnp.int32) - 2
    scale = jnp.take(jnp.array((0.25, 0.125, 0.0625, 0.03125), jnp.float32), k)
    wlv = jnp.take(jnp.array([128, 64, 32, 16], jnp.int32), k)
    lvl_off = jnp.take(jnp.array([0, 32768, 40960, 43008], jnp.int32), k)
    base = lvl_off + b_idx * wlv * wlv
    bx1 = boxes[:, 0] * scale - 0.5
    by1 = boxes[:, 1] * scale - 0.5
    bx2 = boxes[:, 2] * scale - 0.5
    by2 = boxes[:, 3] * scale - 0.5
    bw = (bx2 - bx1) / _OUT
    bh = (by2 - by1) / _OUT

    mesh = plsc.VectorSubcoreMesh(core_axis_name="c", subcore_axis_name="s")
    out_t = pl.kernel(
        _sc_body,
        out_type=jax.ShapeDtypeStruct((512, 49, _C), jnp.float32),
        mesh=mesh,
        scratch_types=[
            pltpu.VMEM((16,), jnp.float32),
            pltpu.VMEM((16,), jnp.float32),
            pltpu.VMEM((16,), jnp.float32),
            pltpu.VMEM((16,), jnp.float32),
            pltpu.VMEM((16,), jnp.int32),
            pltpu.VMEM((16,), jnp.int32),
            pltpu.VMEM((128,), jnp.int32),
            pltpu.VMEM((128,), jnp.int32),
            pltpu.VMEM((8, 16), jnp.float32),
            pltpu.VMEM((8, 16), jnp.float32),
            pltpu.VMEM((128, _C // 2), jnp.int32),
            pltpu.VMEM((128, _C // 2), jnp.int32),
            pltpu.VMEM((49, _C), jnp.float32),
            pltpu.SemaphoreType.DMA,
            pltpu.SemaphoreType.DMA,
        ],
    )(table, bx1, by1, bw, bh, base, wlv)

    return out_t.reshape(512, _OUT, _OUT, _C).transpose(0, 3, 1, 2)
